# Initial kernel scaffold; baseline (speedup 1.0000x reference)
import jax
import jax.numpy as jnp
from jax.experimental import pallas as pl


def kernel(Z, rbf, neighbors, params):
    raise NotImplementedError("write your pallas kernel here")


# speedup vs baseline: 1.9609x; 1.9609x over previous
import functools

import jax
import jax.numpy as jnp
from jax import lax
from jax.experimental import pallas as pl
from jax.experimental.pallas import tpu as pltpu
from jax.experimental.pallas import tpu_sc as plsc

_LN2 = 0.6931471805599453
_TN = 200


def _ssp(x):
    m = jnp.maximum(x, 0.0)
    return m + jnp.log(jnp.exp(x - m) + jnp.exp(-m)) - _LN2


def _dot(a, b):
    return jax.lax.dot_general(a, b, (((a.ndim - 1,), (0,)), ((), ())),
                               preferred_element_type=jnp.float32)


def _sc_gather(table, idx):
    V, Fd = table.shape
    B = idx.shape[0]
    try:
        info = plsc.get_sparse_core_info()
        nc, ns = info.num_cores, info.num_subcores
    except Exception:
        nc, ns = 2, 16
    nw = nc * ns
    per = B // nw
    assert per * nw == B and per % 8 == 0
    ch = 128
    full = per // ch
    tail = per - full * ch
    mesh = plsc.VectorSubcoreMesh(core_axis_name="c", subcore_axis_name="s")

    @functools.partial(
        pl.kernel, mesh=mesh,
        out_type=jax.ShapeDtypeStruct((B, Fd), jnp.float32),
        scratch_types=[
            pltpu.VMEM((ch,), jnp.int32),
            pltpu.VMEM((ch, Fd), jnp.float32),
            pltpu.SemaphoreType.DMA,
        ],
    )
    def gather(table_hbm, idx_hbm, out_hbm, idx_v, rows_v, sem):
        wid = lax.axis_index("s") * nc + lax.axis_index("c")
        base = wid * per

        def step(j, carry):
            off = base + j * ch
            pltpu.sync_copy(idx_hbm.at[pl.ds(off, ch)], idx_v)
            pltpu.async_copy(table_hbm.at[idx_v], rows_v, sem).wait()
            pltpu.sync_copy(rows_v, out_hbm.at[pl.ds(off, ch)])
            return carry

        lax.fori_loop(0, full, step, 0)
        if tail:
            off = base + full * ch
            pltpu.sync_copy(idx_hbm.at[pl.ds(off, tail)],
                            idx_v.at[pl.ds(0, tail)])
            pltpu.async_copy(table_hbm.at[idx_v.at[pl.ds(0, tail)]],
                             rows_v.at[pl.ds(0, tail)], sem).wait()
            pltpu.sync_copy(rows_v.at[pl.ds(0, tail)],
                            out_hbm.at[pl.ds(off, tail)])

    return gather(table, idx)


def _embed_call(Zf, emb, w_in, b_in):
    N = Zf.shape[0]
    A, Fd = emb.shape
    grid = N // _TN

    def body(z_ref, emb_ref, wi_ref, bi_ref, x_ref, v_ref):
        ar = lax.broadcasted_iota(jnp.int32, (_TN, A), 1)
        onehot = (ar == z_ref[...]).astype(jnp.float32)
        x = _dot(onehot, emb_ref[...])
        x_ref[...] = x
        v_ref[...] = _dot(x, wi_ref[...]) + bi_ref[...]

    return pl.pallas_call(
        body,
        grid=(grid,),
        in_specs=[
            pl.BlockSpec((_TN, 1), lambda i: (i, 0)),
            pl.BlockSpec((A, Fd), lambda i: (0, 0)),
            pl.BlockSpec((Fd, Fd), lambda i: (0, 0)),
            pl.BlockSpec((1, Fd), lambda i: (0, 0)),
        ],
        out_specs=[
            pl.BlockSpec((_TN, Fd), lambda i: (i, 0)),
            pl.BlockSpec((_TN, Fd), lambda i: (i, 0)),
        ],
        out_shape=[
            jax.ShapeDtypeStruct((N, Fd), jnp.float32),
            jax.ShapeDtypeStruct((N, Fd), jnp.float32),
        ],
    )(Zf, emb, w_in, b_in)


def _interaction_call(x, vj, rbf2, blk, nxt, K):
    N, Fd = x.shape
    R = rbf2.shape[1]
    grid = N // _TN
    rows = _TN * K

    def body(*refs):
        (rbf_ref, vj_ref, x_ref, fw1, fb1, fw2, fb2, w1, b1, w2, b2) = refs[:11]
        rest = refs[11:]
        u = _ssp(_dot(rbf_ref[...], fw1[...]) + fb1[...])
        w = _dot(u, fw2[...]) + fb2[...]
        p = w * vj_ref[...]
        y = p.reshape(_TN, K, Fd).sum(axis=1)
        y = _ssp(_dot(y, w1[...]) + b1[...])
        y = _dot(y, w2[...]) + b2[...]
        xo = x_ref[...] + y
        if nxt is not None:
            wi, bi, xo_ref, vn_ref = rest
            xo_ref[...] = xo
            vn_ref[...] = _dot(xo, wi[...]) + bi[...]
        else:
            (xo_ref,) = rest
            xo_ref[...] = xo

    wspec = lambda s: pl.BlockSpec(s, lambda i: (0, 0))
    in_specs = [
        pl.BlockSpec((rows, R), lambda i: (i, 0)),
        pl.BlockSpec((rows, Fd), lambda i: (i, 0)),
        pl.BlockSpec((_TN, Fd), lambda i: (i, 0)),
        wspec((R, Fd)), wspec((1, Fd)), wspec((Fd, Fd)), wspec((1, Fd)),
        wspec((Fd, Fd)), wspec((1, Fd)), wspec((Fd, Fd)), wspec((1, Fd)),
    ]
    args = [rbf2, vj, x,
            blk["fw1"], blk["fb1"].reshape(1, Fd),
            blk["fw2"], blk["fb2"].reshape(1, Fd),
            blk["w1"], blk["b1"].reshape(1, Fd),
            blk["w2"], blk["b2"].reshape(1, Fd)]
    out_specs = [pl.BlockSpec((_TN, Fd), lambda i: (i, 0))]
    out_shape = [jax.ShapeDtypeStruct((N, Fd), jnp.float32)]
    if nxt is not None:
        in_specs += [wspec((Fd, Fd)), wspec((1, Fd))]
        args += [nxt["w_in"], nxt["b_in"].reshape(1, Fd)]
        out_specs.append(pl.BlockSpec((_TN, Fd), lambda i: (i, 0)))
        out_shape.append(jax.ShapeDtypeStruct((N, Fd), jnp.float32))

    out = pl.pallas_call(
        body, grid=(grid,), in_specs=in_specs, out_specs=out_specs,
        out_shape=out_shape,
    )(*args)
    return (out[0], out[1]) if nxt is not None else (out[0], None)


def _readout_call(x, ro):
    N, Fd = x.shape
    H = ro["rw1"].shape[1]
    grid = N // _TN

    def body(x_ref, rw1, rb1, rw2, rb2, out_ref, acc_ref):
        i = pl.program_id(0)

        @pl.when(i == 0)
        def _():
            acc_ref[...] = jnp.zeros_like(acc_ref)

        h = _ssp(_dot(x_ref[...], rw1[...]) + rb1[...])
        acc_ref[...] += jnp.sum(h, axis=0, keepdims=True)

        @pl.when(i == grid - 1)
        def _():
            out_ref[...] = _dot(acc_ref[...], rw2[...]) + N * rb2[...]

    wspec = lambda s: pl.BlockSpec(s, lambda i: (0, 0))
    return pl.pallas_call(
        body,
        grid=(grid,),
        in_specs=[
            pl.BlockSpec((_TN, Fd), lambda i: (i, 0)),
            wspec((Fd, H)), wspec((1, H)), wspec((H, 1)), wspec((1, 1)),
        ],
        out_specs=pl.BlockSpec((1, 1), lambda i: (0, 0)),
        out_shape=jax.ShapeDtypeStruct((1, 1), jnp.float32),
        scratch_shapes=[pltpu.VMEM((1, H), jnp.float32)],
        compiler_params=pltpu.CompilerParams(
            dimension_semantics=("arbitrary",)),
    )(x, ro["rw1"], ro["rb1"].reshape(1, H), ro["rw2"],
      ro["rb2"].reshape(1, 1))


def kernel(Z, rbf, neighbors, params):
    emb = params["embedding"]
    blocks = params["blocks"]
    ro = params["readout"]
    N, K = neighbors.shape
    R = rbf.shape[-1]
    Fd = emb.shape[1]

    rbf2 = rbf.reshape(N * K, R)
    nbr = neighbors.reshape(N * K).astype(jnp.int32)
    Zf = Z.astype(jnp.int32).reshape(N, 1)

    x, v = _embed_call(Zf, emb, blocks[0]["w_in"],
                       blocks[0]["b_in"].reshape(1, Fd))
    for t in range(len(blocks)):
        vj = _sc_gather(v, nbr)
        nxt = blocks[t + 1] if t + 1 < len(blocks) else None
        x, v = _interaction_call(x, vj, rbf2, blocks[t], nxt, K)
    e = _readout_call(x, ro)
    return e.reshape(())


# speedup vs baseline: 2.8259x; 1.4411x over previous
import functools

import jax
import jax.numpy as jnp
from jax import lax
from jax.experimental import pallas as pl
from jax.experimental.pallas import tpu as pltpu
from jax.experimental.pallas import tpu_sc as plsc

_LN2 = 0.6931471805599453
_TN = 200


def _ssp(x):
    m = jnp.maximum(x, 0.0)
    return m + jnp.log(jnp.exp(x - m) + jnp.exp(-m)) - _LN2


def _dot(a, b):
    return jax.lax.dot_general(a, b, (((a.ndim - 1,), (0,)), ((), ())),
                               preferred_element_type=jnp.float32)


def _sc_gather(table, idx):
    V, Fd = table.shape
    B = idx.shape[0]
    try:
        info = plsc.get_sparse_core_info()
        nc, ns = info.num_cores, info.num_subcores
    except Exception:
        nc, ns = 2, 16
    nw = nc * ns
    per = B // nw
    assert per * nw == B and per % 8 == 0
    ch = 128
    nbuf = 4
    full = per // ch
    groups = full // nbuf
    rest = full - groups * nbuf
    tail = per - full * ch
    mesh = plsc.VectorSubcoreMesh(core_axis_name="c", subcore_axis_name="s")

    @functools.partial(
        pl.kernel, mesh=mesh,
        out_type=jax.ShapeDtypeStruct((B, Fd), jnp.float32),
        scratch_types=[
            pltpu.VMEM((per,), jnp.int32),
            pltpu.VMEM((nbuf, ch, Fd), jnp.float32),
            pltpu.SemaphoreType.DMA,
            pltpu.SemaphoreType.DMA,
        ],
    )
    def gather(table_hbm, idx_hbm, out_hbm, idx_v, rows_v, sem_g, sem_w):
        wid = lax.axis_index("s") * nc + lax.axis_index("c")
        base = wid * per
        pltpu.sync_copy(idx_hbm.at[pl.ds(base, per)], idx_v)

        def group(g, carry):
            off0 = g * nbuf * ch

            @pl.when(g > 0)
            def _():
                for b in range(nbuf):
                    pltpu.make_async_copy(
                        rows_v.at[b],
                        out_hbm.at[pl.ds(base + off0 + b * ch, ch)],
                        sem_w).wait()

            for b in range(nbuf):
                pltpu.async_copy(
                    table_hbm.at[idx_v.at[pl.ds(off0 + b * ch, ch)]],
                    rows_v.at[b], sem_g)

            for b in range(nbuf):
                pltpu.make_async_copy(
                    table_hbm.at[idx_v.at[pl.ds(off0 + b * ch, ch)]],
                    rows_v.at[b], sem_g).wait()
                pltpu.async_copy(
                    rows_v.at[b],
                    out_hbm.at[pl.ds(base + off0 + b * ch, ch)], sem_w)
            return carry

        lax.fori_loop(0, groups, group, 0)
        for b in range(nbuf):
            pltpu.make_async_copy(
                rows_v.at[b], out_hbm.at[pl.ds(base, ch)], sem_w).wait()
        for r in range(rest):
            off = (groups * nbuf + r) * ch
            pltpu.async_copy(table_hbm.at[idx_v.at[pl.ds(off, ch)]],
                             rows_v.at[0], sem_g).wait()
            pltpu.sync_copy(rows_v.at[0], out_hbm.at[pl.ds(base + off, ch)])
        if tail:
            off = full * ch
            pltpu.async_copy(
                table_hbm.at[idx_v.at[pl.ds(off, tail)]],
                rows_v.at[0].at[pl.ds(0, tail)], sem_g).wait()
            pltpu.sync_copy(rows_v.at[0].at[pl.ds(0, tail)],
                            out_hbm.at[pl.ds(base + off, tail)])

    return gather(table, idx)


def _embed_call(Zf, emb, w_in, b_in):
    N = Zf.shape[0]
    A, Fd = emb.shape
    grid = N // _TN

    def body(z_ref, emb_ref, wi_ref, bi_ref, x_ref, v_ref):
        ar = lax.broadcasted_iota(jnp.int32, (_TN, A), 1)
        onehot = (ar == z_ref[...]).astype(jnp.float32)
        x = _dot(onehot, emb_ref[...])
        x_ref[...] = x
        v_ref[...] = _dot(x, wi_ref[...]) + bi_ref[...]

    return pl.pallas_call(
        body,
        grid=(grid,),
        in_specs=[
            pl.BlockSpec((_TN, 1), lambda i: (i, 0)),
            pl.BlockSpec((A, Fd), lambda i: (0, 0)),
            pl.BlockSpec((Fd, Fd), lambda i: (0, 0)),
            pl.BlockSpec((1, Fd), lambda i: (0, 0)),
        ],
        out_specs=[
            pl.BlockSpec((_TN, Fd), lambda i: (i, 0)),
            pl.BlockSpec((_TN, Fd), lambda i: (i, 0)),
        ],
        out_shape=[
            jax.ShapeDtypeStruct((N, Fd), jnp.float32),
            jax.ShapeDtypeStruct((N, Fd), jnp.float32),
        ],
    )(Zf, emb, w_in, b_in)


def _interaction_call(x, vj, rbf3, blk, nxt, K):
    N, Fd = x.shape
    R = rbf3.shape[-1]
    grid = N // _TN
    rows = _TN * K

    def body(*refs):
        (rbf_ref, vj_ref, x_ref, fw1, fb1, fw2, fb2, w1, b1, w2, b2) = refs[:11]
        rest = refs[11:]
        u = _ssp(_dot(rbf_ref[...].reshape(rows, R), fw1[...]) + fb1[...])
        w = _dot(u, fw2[...]) + fb2[...]
        p = w * vj_ref[...]
        y = p.reshape(_TN, K, Fd).sum(axis=1)
        y = _ssp(_dot(y, w1[...]) + b1[...])
        y = _dot(y, w2[...]) + b2[...]
        xo = x_ref[...] + y
        if nxt is not None:
            wi, bi, xo_ref, vn_ref = rest
            xo_ref[...] = xo
            vn_ref[...] = _dot(xo, wi[...]) + bi[...]
        else:
            (xo_ref,) = rest
            xo_ref[...] = xo

    wspec = lambda s: pl.BlockSpec(s, lambda i: (0, 0))
    in_specs = [
        pl.BlockSpec((_TN, K, R), lambda i: (i, 0, 0)),
        pl.BlockSpec((rows, Fd), lambda i: (i, 0)),
        pl.BlockSpec((_TN, Fd), lambda i: (i, 0)),
        wspec((R, Fd)), wspec((1, Fd)), wspec((Fd, Fd)), wspec((1, Fd)),
        wspec((Fd, Fd)), wspec((1, Fd)), wspec((Fd, Fd)), wspec((1, Fd)),
    ]
    args = [rbf3, vj, x,
            blk["fw1"], blk["fb1"].reshape(1, Fd),
            blk["fw2"], blk["fb2"].reshape(1, Fd),
            blk["w1"], blk["b1"].reshape(1, Fd),
            blk["w2"], blk["b2"].reshape(1, Fd)]
    out_specs = [pl.BlockSpec((_TN, Fd), lambda i: (i, 0))]
    out_shape = [jax.ShapeDtypeStruct((N, Fd), jnp.float32)]
    if nxt is not None:
        in_specs += [wspec((Fd, Fd)), wspec((1, Fd))]
        args += [nxt["w_in"], nxt["b_in"].reshape(1, Fd)]
        out_specs.append(pl.BlockSpec((_TN, Fd), lambda i: (i, 0)))
        out_shape.append(jax.ShapeDtypeStruct((N, Fd), jnp.float32))

    out = pl.pallas_call(
        body, grid=(grid,), in_specs=in_specs, out_specs=out_specs,
        out_shape=out_shape,
    )(*args)
    return (out[0], out[1]) if nxt is not None else (out[0], None)


def _readout_call(x, ro):
    N, Fd = x.shape
    H = ro["rw1"].shape[1]
    grid = N // _TN

    def body(x_ref, rw1, rb1, rw2, rb2, out_ref, acc_ref):
        i = pl.program_id(0)

        @pl.when(i == 0)
        def _():
            acc_ref[...] = jnp.zeros_like(acc_ref)

        h = _ssp(_dot(x_ref[...], rw1[...]) + rb1[...])
        acc_ref[...] += jnp.sum(h, axis=0, keepdims=True)

        @pl.when(i == grid - 1)
        def _():
            out_ref[...] = _dot(acc_ref[...], rw2[...]) + N * rb2[...]

    wspec = lambda s: pl.BlockSpec(s, lambda i: (0, 0))
    return pl.pallas_call(
        body,
        grid=(grid,),
        in_specs=[
            pl.BlockSpec((_TN, Fd), lambda i: (i, 0)),
            wspec((Fd, H)), wspec((1, H)), wspec((H, 1)), wspec((1, 1)),
        ],
        out_specs=pl.BlockSpec((1, 1), lambda i: (0, 0)),
        out_shape=jax.ShapeDtypeStruct((1, 1), jnp.float32),
        scratch_shapes=[pltpu.VMEM((1, H), jnp.float32)],
        compiler_params=pltpu.CompilerParams(
            dimension_semantics=("arbitrary",)),
    )(x, ro["rw1"], ro["rb1"].reshape(1, H), ro["rw2"],
      ro["rb2"].reshape(1, 1))


def kernel(Z, rbf, neighbors, params):
    emb = params["embedding"]
    blocks = params["blocks"]
    ro = params["readout"]
    N, K = neighbors.shape
    R = rbf.shape[-1]
    Fd = emb.shape[1]

    nbr = neighbors.reshape(N * K).astype(jnp.int32)
    Zf = Z.astype(jnp.int32).reshape(N, 1)

    x, v = _embed_call(Zf, emb, blocks[0]["w_in"],
                       blocks[0]["b_in"].reshape(1, Fd))
    for t in range(len(blocks)):
        vj = _sc_gather(v, nbr)
        nxt = blocks[t + 1] if t + 1 < len(blocks) else None
        x, v = _interaction_call(x, vj, rbf, blocks[t], nxt, K)
    e = _readout_call(x, ro)
    return e.reshape(())
